# baseline (speedup 1.0000x reference)
import functools

import jax
import jax.numpy as jnp
from jax import lax
from jax.experimental import pallas as pl
from jax.experimental.pallas import tpu as pltpu
from jax.experimental.pallas import tpu_sc as plsc

N = 10000
E = 320000
D = 128
NG = 64

NC = 2
NS = 16
NW = NC * NS

LANES = 128
EPT_ROWS = 80
EPT = EPT_ROWS * LANES
E_PAD = NW * EPT
ACC_ROWS = 10112
RPT = ACC_ROWS // NS
TRASH = N

_MESH = plsc.VectorSubcoreMesh(core_axis_name="c", subcore_axis_name="s")


_NBUF = 3
_SCATTER_OUT = jax.ShapeDtypeStruct((NC, ACC_ROWS, D), jnp.float32)
_SCATTER_SCRATCH = [
    [pltpu.VMEM((LANES,), jnp.int32) for _ in range(_NBUF)],
    [pltpu.VMEM((LANES,), jnp.int32) for _ in range(_NBUF)],
    [pltpu.VMEM((LANES, D), jnp.float32) for _ in range(_NBUF)],
    pltpu.MemorySpace.VMEM_SHARED((ACC_ROWS, D), jnp.float32),
    [pltpu.SemaphoreType.DMA for _ in range(_NBUF)],
    [pltpu.SemaphoreType.DMA for _ in range(_NBUF)],
]


def _sc_scatter_rows_body(g_hbm, src_hbm, dst_hbm, z_hbm, out_hbm,
                          src_b, dst_b, rows, acc, isems, gsems):
    c = lax.axis_index("c")
    s = lax.axis_index("s")
    wid = s * NC + c
    base = wid * EPT

    def fire_idx(j, b):
        pltpu.async_copy(src_hbm.at[pl.ds(base + j * LANES, LANES)],
                         src_b[b], isems[b])
        pltpu.async_copy(dst_hbm.at[pl.ds(base + j * LANES, LANES)],
                         dst_b[b], isems[b])

    def wait_idx(b):
        pltpu.make_async_copy(src_hbm.at[pl.ds(base, LANES)],
                              src_b[b], isems[b]).wait()
        pltpu.make_async_copy(dst_hbm.at[pl.ds(base, LANES)],
                              dst_b[b], isems[b]).wait()

    def fire_gather(b):
        pltpu.async_copy(g_hbm.at[src_b[b]], rows[b], gsems[b])

    def wait_gather(b):
        pltpu.make_async_copy(g_hbm.at[src_b[b]], rows[b], gsems[b]).wait()

    for b in range(_NBUF):
        fire_idx(b, b)
    pltpu.sync_copy(z_hbm, acc.at[pl.ds(s * RPT, RPT)])
    wait_idx(0)
    fire_gather(0)
    wait_idx(1)
    fire_gather(1)
    plsc.subcore_barrier()

    def step(j, b):
        wait_idx((b + 2) % _NBUF)
        fire_gather((b + 2) % _NBUF)
        wait_gather(b)
        pltpu.sync_copy(rows[b], acc.at[dst_b[b]], add=True)
        fire_idx(j + _NBUF, b)

    def body(k, carry):
        j0 = k * _NBUF
        for b in range(_NBUF):
            step(j0 + b, b)
        return carry

    lax.fori_loop(0, (EPT_ROWS - 5) // _NBUF, body, 0, unroll=False)
    for j in range(EPT_ROWS - 5, EPT_ROWS):
        b = j % _NBUF
        if j + 2 < EPT_ROWS:
            wait_idx((b + 2) % _NBUF)
            fire_gather((b + 2) % _NBUF)
        wait_gather(b)
        pltpu.sync_copy(rows[b], acc.at[dst_b[b]], add=True)
        if j + _NBUF < EPT_ROWS:
            fire_idx(j + _NBUF, b)
    plsc.subcore_barrier()
    pltpu.sync_copy(acc.at[pl.ds(s * RPT, RPT)],
                    out_hbm.at[c].at[pl.ds(s * RPT, RPT)])


_sc_scatter_rows = functools.partial(
    pl.kernel, out_type=_SCATTER_OUT, mesh=_MESH,
    scratch_types=_SCATTER_SCRATCH)(_sc_scatter_rows_body)


_DEG_OUT = jax.ShapeDtypeStruct((NC, ACC_ROWS, D), jnp.float32)
_DEG_SCRATCH = [
    [pltpu.VMEM((LANES,), jnp.int32) for _ in range(_NBUF)],
    pltpu.VMEM((LANES, D), jnp.float32),
    pltpu.MemorySpace.VMEM_SHARED((ACC_ROWS, D), jnp.float32),
    [pltpu.SemaphoreType.DMA for _ in range(_NBUF)],
]


def _sc_degree_body(dst_hbm, ones_hbm, z_hbm, out_hbm, dst_b, ones_v, acc,
                    isems):
    c = lax.axis_index("c")
    s = lax.axis_index("s")
    wid = s * NC + c
    base = wid * EPT

    def fire_idx(j, b):
        pltpu.async_copy(dst_hbm.at[pl.ds(base + j * LANES, LANES)],
                         dst_b[b], isems[b])

    def wait_idx(b):
        pltpu.make_async_copy(dst_hbm.at[pl.ds(base, LANES)],
                              dst_b[b], isems[b]).wait()

    for b in range(_NBUF):
        fire_idx(b, b)
    pltpu.sync_copy(z_hbm, acc.at[pl.ds(s * RPT, RPT)])
    pltpu.sync_copy(ones_hbm, ones_v)
    plsc.subcore_barrier()

    def body(k, carry):
        j0 = k * _NBUF
        for b in range(_NBUF):
            wait_idx(b)
            pltpu.sync_copy(ones_v, acc.at[dst_b[b]], add=True)
            fire_idx(j0 + b + _NBUF, b)
        return carry

    lax.fori_loop(0, (EPT_ROWS - 5) // _NBUF, body, 0, unroll=False)
    for j in range(EPT_ROWS - 5, EPT_ROWS):
        b = j % _NBUF
        wait_idx(b)
        pltpu.sync_copy(ones_v, acc.at[dst_b[b]], add=True)
        if j + _NBUF < EPT_ROWS:
            fire_idx(j + _NBUF, b)
    plsc.subcore_barrier()
    pltpu.sync_copy(acc.at[pl.ds(s * RPT, RPT)],
                    out_hbm.at[c].at[pl.ds(s * RPT, RPT)])


_sc_degree = functools.partial(
    pl.kernel, out_type=_DEG_OUT, mesh=_MESH,
    scratch_types=_DEG_SCRATCH)(_sc_degree_body)


_BLK = 1000


def _dinv_block(dega_ref, degb_ref):
    deg = dega_ref[:, 0:1] + degb_ref[:, 0:1] + 1.0
    return lax.rsqrt(deg)


def _tc_g1_body(dega_ref, degb_ref, x_ref, w_ref, g_ref):
    dinv = _dinv_block(dega_ref, degb_ref)
    g_ref[...] = dinv * jnp.dot(x_ref[...], w_ref[...],
                                preferred_element_type=jnp.float32)


def _tc_g1(dega, degb, x, W1):
    return pl.pallas_call(
        _tc_g1_body,
        grid=(N // _BLK,),
        in_specs=[
            pl.BlockSpec((_BLK, 16), lambda i: (i, 0)),
            pl.BlockSpec((_BLK, 16), lambda i: (i, 0)),
            pl.BlockSpec((_BLK, D), lambda i: (i, 0)),
            pl.BlockSpec((D, D), lambda i: (0, 0)),
        ],
        out_specs=pl.BlockSpec((_BLK, D), lambda i: (i, 0)),
        out_shape=jax.ShapeDtypeStruct((N, D), jnp.float32),
    )(dega, degb, x, W1)


def _tc_g2_body(dega_ref, degb_ref, s0_ref, s1_ref, g1_ref, b1_ref, w2_ref,
                g2_ref):
    dinv = _dinv_block(dega_ref, degb_ref)
    h1 = dinv * (s0_ref[...] + s1_ref[...] + g1_ref[...]) + b1_ref[...]
    h1 = jnp.maximum(h1, 0.0)
    g2_ref[...] = dinv * jnp.dot(h1, w2_ref[...],
                                 preferred_element_type=jnp.float32)


def _tc_g2(dega, degb, s0, s1, g1, b1r, W2):
    return pl.pallas_call(
        _tc_g2_body,
        grid=(N // _BLK,),
        in_specs=[
            pl.BlockSpec((_BLK, 16), lambda i: (i, 0)),
            pl.BlockSpec((_BLK, 16), lambda i: (i, 0)),
            pl.BlockSpec((_BLK, D), lambda i: (i, 0)),
            pl.BlockSpec((_BLK, D), lambda i: (i, 0)),
            pl.BlockSpec((_BLK, D), lambda i: (i, 0)),
            pl.BlockSpec((1, D), lambda i: (0, 0)),
            pl.BlockSpec((D, D), lambda i: (0, 0)),
        ],
        out_specs=pl.BlockSpec((_BLK, D), lambda i: (i, 0)),
        out_shape=jax.ShapeDtypeStruct((N, D), jnp.float32),
    )(dega, degb, s0, s1, g1, b1r, W2)


def _tc_head_body(dega_ref, degb_ref, s0_ref, s1_ref, g2_ref, b2_ref,
                  batch_ref, wm1_ref, bm1_ref, wm2_ref, bm2_ref, out_ref):
    deg = dega_ref[:, 0:1] + degb_ref[:, 0:1] + 1.0
    dinv = lax.rsqrt(deg)
    h2 = dinv * (s0_ref[...] + s1_ref[...] + g2_ref[...]) + b2_ref[...]
    gid = lax.broadcasted_iota(jnp.int32, (NG, N), 0).astype(jnp.float32)
    onehot = (batch_ref[...] == gid).astype(jnp.float32)
    sums = jnp.dot(onehot, h2, preferred_element_type=jnp.float32)
    counts = jnp.sum(onehot, axis=1, keepdims=True)
    pooled = sums / jnp.maximum(counts, 1.0)
    z = jnp.dot(pooled, wm1_ref[...], preferred_element_type=jnp.float32)
    z = jnp.maximum(z + bm1_ref[...], 0.0)
    out_ref[...] = (jnp.sum(z * wm2_ref[...], axis=1, keepdims=True)
                    + bm2_ref[...])


def _tc_head(dega, degb, s0, s1, g2, b2r, batchf, Wm1, bm1r, wm2r, bm2r):
    return pl.pallas_call(
        _tc_head_body,
        out_shape=jax.ShapeDtypeStruct((NG, 1), jnp.float32),
    )(dega, degb, s0, s1, g2, b2r, batchf, Wm1, bm1r, wm2r, bm2r)


def kernel(x, edge_index, batch, W1, b1, W2, b2, Wm1, bm1, Wm2, bm2):
    src = edge_index[0]
    dst = edge_index[1]
    pad = E_PAD - E
    src_p = jnp.concatenate([src, jnp.zeros((pad,), jnp.int32)])
    dst_p = jnp.concatenate([dst, jnp.full((pad,), TRASH, jnp.int32)])
    zeros_d = jnp.zeros((RPT, D), jnp.float32)
    ones_d = jnp.ones((LANES, D), jnp.float32)

    degp = _sc_degree(dst_p, ones_d, zeros_d)
    dega = degp[0, :N, 0:16]
    degb = degp[1, :N, 0:16]

    g1 = _tc_g1(dega, degb, x, W1)
    S1 = _sc_scatter_rows(g1, src_p, dst_p, zeros_d)
    g2 = _tc_g2(dega, degb, S1[0, :N], S1[1, :N], g1,
                b1.reshape(1, D), W2)
    S2 = _sc_scatter_rows(g2, src_p, dst_p, zeros_d)
    out = _tc_head(dega, degb, S2[0, :N], S2[1, :N], g2,
                   b2.reshape(1, D),
                   batch.astype(jnp.float32).reshape(1, N),
                   Wm1, bm1.reshape(1, 16),
                   Wm2.reshape(1, 16), bm2.reshape(1, 1))
    return out.reshape(-1)


# speedup vs baseline: 3.2414x; 3.2414x over previous
import functools

import jax
import jax.numpy as jnp
from jax import lax
from jax.experimental import pallas as pl
from jax.experimental.pallas import tpu as pltpu
from jax.experimental.pallas import tpu_sc as plsc

N = 10000
E = 320000
D = 128
NG = 64

NC = 2
NS = 16
NW = NC * NS

LANES = 128
STEPS = 80
EPT = STEPS * LANES
E_PAD = NW * EPT
ACC_ROWS = 10112
RPT = ACC_ROWS // NS
TRASH = N
N_TRASH = ACC_ROWS - N

_MESH = plsc.VectorSubcoreMesh(core_axis_name="c", subcore_axis_name="s")


_NBUF = 3
_SCATTER_OUT = jax.ShapeDtypeStruct((NC, ACC_ROWS, D), jnp.float32)
_SCATTER_SCRATCH = [
    [pltpu.VMEM((LANES,), jnp.int32) for _ in range(_NBUF)],
    [pltpu.VMEM((LANES,), jnp.int32) for _ in range(_NBUF)],
    [pltpu.VMEM((LANES, D), jnp.float32) for _ in range(_NBUF)],
    pltpu.MemorySpace.VMEM_SHARED((ACC_ROWS, D), jnp.float32),
    [pltpu.SemaphoreType.DMA for _ in range(_NBUF)],
    [pltpu.SemaphoreType.DMA for _ in range(_NBUF)],
]


def _sc_scatter_rows_body(g_hbm, src_hbm, dst_hbm, z_hbm, out_hbm,
                          src_b, dst_b, rows, acc, isems, gsems):
    c = lax.axis_index("c")
    s = lax.axis_index("s")
    wid = s * NC + c
    base = wid * EPT

    def fire_idx(j, b):
        pltpu.async_copy(src_hbm.at[pl.ds(base + j * LANES, LANES)],
                         src_b[b], isems[b])
        pltpu.async_copy(dst_hbm.at[pl.ds(base + j * LANES, LANES)],
                         dst_b[b], isems[b])

    def wait_idx(b):
        pltpu.make_async_copy(src_hbm.at[pl.ds(base, LANES)],
                              src_b[b], isems[b]).wait()
        pltpu.make_async_copy(dst_hbm.at[pl.ds(base, LANES)],
                              dst_b[b], isems[b]).wait()

    def fire_gather(b):
        pltpu.async_copy(g_hbm.at[src_b[b]], rows[b], gsems[b])

    def wait_gather(b):
        pltpu.make_async_copy(g_hbm.at[src_b[b]], rows[b], gsems[b]).wait()

    for b in range(_NBUF):
        fire_idx(b, b)
    pltpu.sync_copy(z_hbm, acc.at[pl.ds(s * RPT, RPT)])
    wait_idx(0)
    fire_gather(0)
    wait_idx(1)
    fire_gather(1)
    plsc.subcore_barrier()

    def step(j, b):
        wait_idx((b + 2) % _NBUF)
        fire_gather((b + 2) % _NBUF)
        wait_gather(b)
        pltpu.sync_copy(rows[b], acc.at[dst_b[b]], add=True)
        fire_idx(j + _NBUF, b)

    def body(k, carry):
        j0 = k * _NBUF
        for b in range(_NBUF):
            step(j0 + b, b)
        return carry

    n_main = (STEPS - 4) // _NBUF
    lax.fori_loop(0, n_main, body, 0, unroll=False)
    for j in range(n_main * _NBUF, STEPS):
        b = j % _NBUF
        if j + 2 < STEPS:
            wait_idx((b + 2) % _NBUF)
            fire_gather((b + 2) % _NBUF)
        wait_gather(b)
        pltpu.sync_copy(rows[b], acc.at[dst_b[b]], add=True)
        if j + _NBUF < STEPS:
            fire_idx(j + _NBUF, b)
    plsc.subcore_barrier()
    pltpu.sync_copy(acc.at[pl.ds(s * RPT, RPT)],
                    out_hbm.at[c].at[pl.ds(s * RPT, RPT)])


_sc_scatter_rows = functools.partial(
    pl.kernel, out_type=_SCATTER_OUT, mesh=_MESH,
    scratch_types=_SCATTER_SCRATCH)(_sc_scatter_rows_body)


_DEG_OUT = jax.ShapeDtypeStruct((NC, ACC_ROWS, D), jnp.float32)
_DEG_SCRATCH = [
    [pltpu.VMEM((LANES,), jnp.int32) for _ in range(_NBUF)],
    pltpu.VMEM((LANES, D), jnp.float32),
    pltpu.MemorySpace.VMEM_SHARED((ACC_ROWS, D), jnp.float32),
    [pltpu.SemaphoreType.DMA for _ in range(_NBUF)],
]


def _sc_degree_body(dst_hbm, ones_hbm, z_hbm, out_hbm, dst_b, ones_v, acc,
                    isems):
    c = lax.axis_index("c")
    s = lax.axis_index("s")
    wid = s * NC + c
    base = wid * EPT

    def fire_idx(j, b):
        pltpu.async_copy(dst_hbm.at[pl.ds(base + j * LANES, LANES)],
                         dst_b[b], isems[b])

    def wait_idx(b):
        pltpu.make_async_copy(dst_hbm.at[pl.ds(base, LANES)],
                              dst_b[b], isems[b]).wait()

    for b in range(_NBUF):
        fire_idx(b, b)
    pltpu.sync_copy(z_hbm, acc.at[pl.ds(s * RPT, RPT)])
    pltpu.sync_copy(ones_hbm, ones_v)
    plsc.subcore_barrier()

    def body(k, carry):
        j0 = k * _NBUF
        for b in range(_NBUF):
            wait_idx(b)
            pltpu.sync_copy(ones_v, acc.at[dst_b[b]], add=True)
            fire_idx(j0 + b + _NBUF, b)
        return carry

    n_main = (STEPS - 3) // _NBUF
    lax.fori_loop(0, n_main, body, 0, unroll=False)
    for j in range(n_main * _NBUF, STEPS):
        b = j % _NBUF
        wait_idx(b)
        pltpu.sync_copy(ones_v, acc.at[dst_b[b]], add=True)
        if j + _NBUF < STEPS:
            fire_idx(j + _NBUF, b)
    plsc.subcore_barrier()
    pltpu.sync_copy(acc.at[pl.ds(s * RPT, RPT)],
                    out_hbm.at[c].at[pl.ds(s * RPT, RPT)])


_sc_degree = functools.partial(
    pl.kernel, out_type=_DEG_OUT, mesh=_MESH,
    scratch_types=_DEG_SCRATCH)(_sc_degree_body)


_BLK = 1000


def _dinv_block(dega_ref, degb_ref):
    deg = dega_ref[:, 0:1] + degb_ref[:, 0:1] + 1.0
    return lax.rsqrt(deg)


def _tc_g1_body(dega_ref, degb_ref, x_ref, w_ref, g_ref):
    dinv = _dinv_block(dega_ref, degb_ref)
    g_ref[...] = dinv * jnp.dot(x_ref[...], w_ref[...],
                                preferred_element_type=jnp.float32)


def _tc_g1(dega, degb, x, W1):
    return pl.pallas_call(
        _tc_g1_body,
        grid=(N // _BLK,),
        in_specs=[
            pl.BlockSpec((_BLK, 16), lambda i: (i, 0)),
            pl.BlockSpec((_BLK, 16), lambda i: (i, 0)),
            pl.BlockSpec((_BLK, D), lambda i: (i, 0)),
            pl.BlockSpec((D, D), lambda i: (0, 0)),
        ],
        out_specs=pl.BlockSpec((_BLK, D), lambda i: (i, 0)),
        out_shape=jax.ShapeDtypeStruct((N, D), jnp.float32),
    )(dega, degb, x, W1)


def _tc_g2_body(dega_ref, degb_ref, s0_ref, s1_ref, g1_ref, b1_ref, w2_ref,
                g2_ref):
    dinv = _dinv_block(dega_ref, degb_ref)
    h1 = dinv * (s0_ref[...] + s1_ref[...] + g1_ref[...]) + b1_ref[...]
    h1 = jnp.maximum(h1, 0.0)
    g2_ref[...] = dinv * jnp.dot(h1, w2_ref[...],
                                 preferred_element_type=jnp.float32)


def _tc_g2(dega, degb, s0, s1, g1, b1r, W2):
    return pl.pallas_call(
        _tc_g2_body,
        grid=(N // _BLK,),
        in_specs=[
            pl.BlockSpec((_BLK, 16), lambda i: (i, 0)),
            pl.BlockSpec((_BLK, 16), lambda i: (i, 0)),
            pl.BlockSpec((_BLK, D), lambda i: (i, 0)),
            pl.BlockSpec((_BLK, D), lambda i: (i, 0)),
            pl.BlockSpec((_BLK, D), lambda i: (i, 0)),
            pl.BlockSpec((1, D), lambda i: (0, 0)),
            pl.BlockSpec((D, D), lambda i: (0, 0)),
        ],
        out_specs=pl.BlockSpec((_BLK, D), lambda i: (i, 0)),
        out_shape=jax.ShapeDtypeStruct((N, D), jnp.float32),
    )(dega, degb, s0, s1, g1, b1r, W2)


def _tc_head_body(dega_ref, degb_ref, s0_ref, s1_ref, g2_ref, b2_ref,
                  batch_ref, wm1_ref, bm1_ref, wm2_ref, bm2_ref, out_ref):
    deg = dega_ref[:, 0:1] + degb_ref[:, 0:1] + 1.0
    dinv = lax.rsqrt(deg)
    h2 = dinv * (s0_ref[...] + s1_ref[...] + g2_ref[...]) + b2_ref[...]
    gid = lax.broadcasted_iota(jnp.int32, (NG, N), 0).astype(jnp.float32)
    onehot = (batch_ref[...] == gid).astype(jnp.float32)
    sums = jnp.dot(onehot, h2, preferred_element_type=jnp.float32)
    counts = jnp.sum(onehot, axis=1, keepdims=True)
    pooled = sums / jnp.maximum(counts, 1.0)
    z = jnp.dot(pooled, wm1_ref[...], preferred_element_type=jnp.float32)
    z = jnp.maximum(z + bm1_ref[...], 0.0)
    out_ref[...] = (jnp.sum(z * wm2_ref[...], axis=1, keepdims=True)
                    + bm2_ref[...])


def _tc_head(dega, degb, s0, s1, g2, b2r, batchf, Wm1, bm1r, wm2r, bm2r):
    return pl.pallas_call(
        _tc_head_body,
        out_shape=jax.ShapeDtypeStruct((NG, 1), jnp.float32),
    )(dega, degb, s0, s1, g2, b2r, batchf, Wm1, bm1r, wm2r, bm2r)


def kernel(x, edge_index, batch, W1, b1, W2, b2, Wm1, bm1, Wm2, bm2):
    src = edge_index[0]
    dst = edge_index[1]
    pad = E_PAD - E
    it = jnp.arange(pad, dtype=jnp.int32)
    src_p = jnp.concatenate([src, it % N])
    dst_p = jnp.concatenate([dst, TRASH + (it % N_TRASH)])
    zeros_d = jnp.zeros((RPT, D), jnp.float32)
    ones_d = jnp.ones((LANES, D), jnp.float32)

    degp = _sc_degree(dst_p, ones_d, zeros_d)
    dega = degp[0, :N, 0:16]
    degb = degp[1, :N, 0:16]

    g1 = _tc_g1(dega, degb, x, W1)
    S1 = _sc_scatter_rows(g1, src_p, dst_p, zeros_d)
    g2 = _tc_g2(dega, degb, S1[0, :N], S1[1, :N], g1,
                b1.reshape(1, D), W2)
    S2 = _sc_scatter_rows(g2, src_p, dst_p, zeros_d)
    out = _tc_head(dega, degb, S2[0, :N], S2[1, :N], g2,
                   b2.reshape(1, D),
                   batch.astype(jnp.float32).reshape(1, N),
                   Wm1, bm1.reshape(1, 16),
                   Wm2.reshape(1, 16), bm2.reshape(1, 1))
    return out.reshape(-1)


# speedup vs baseline: 3.4362x; 1.0601x over previous
import functools

import jax
import jax.numpy as jnp
from jax import lax
from jax.experimental import pallas as pl
from jax.experimental.pallas import tpu as pltpu
from jax.experimental.pallas import tpu_sc as plsc

N = 10000
E = 320000
D = 128
NG = 64

NC = 2
NS = 16
NW = NC * NS

LANES = 128
STEPS = 80
EPT = STEPS * LANES
E_PAD = NW * EPT
ACC_ROWS = 10112
RPT = ACC_ROWS // NS
TRASH = N
N_TRASH = ACC_ROWS - N

_MESH = plsc.VectorSubcoreMesh(core_axis_name="c", subcore_axis_name="s")


_NBUF = 3
_SCATTER_OUT = jax.ShapeDtypeStruct((NC, ACC_ROWS, D), jnp.float32)
_SCATTER_SCRATCH = [
    [pltpu.VMEM((LANES,), jnp.int32) for _ in range(_NBUF)],
    [pltpu.VMEM((LANES,), jnp.int32) for _ in range(_NBUF)],
    [pltpu.VMEM((LANES, D), jnp.float32) for _ in range(_NBUF)],
    pltpu.MemorySpace.VMEM_SHARED((ACC_ROWS, D), jnp.float32),
    [pltpu.SemaphoreType.DMA for _ in range(_NBUF)],
    [pltpu.SemaphoreType.DMA for _ in range(_NBUF)],
]


def _sc_scatter_rows_body(g_hbm, src_hbm, dst_hbm, z_hbm, out_hbm,
                          src_b, dst_b, rows, acc, isems, gsems):
    c = lax.axis_index("c")
    s = lax.axis_index("s")
    wid = s * NC + c
    base = wid * EPT

    def fire_idx(j, b):
        pltpu.async_copy(src_hbm.at[pl.ds(base + j * LANES, LANES)],
                         src_b[b], isems[b])
        pltpu.async_copy(dst_hbm.at[pl.ds(base + j * LANES, LANES)],
                         dst_b[b], isems[b])

    def wait_idx(b):
        pltpu.make_async_copy(src_hbm.at[pl.ds(base, LANES)],
                              src_b[b], isems[b]).wait()
        pltpu.make_async_copy(dst_hbm.at[pl.ds(base, LANES)],
                              dst_b[b], isems[b]).wait()

    def fire_gather(b):
        pltpu.async_copy(g_hbm.at[src_b[b]], rows[b], gsems[b])

    def wait_gather(b):
        pltpu.make_async_copy(g_hbm.at[src_b[b]], rows[b], gsems[b]).wait()

    for b in range(_NBUF):
        fire_idx(b, b)
    pltpu.sync_copy(z_hbm, acc.at[pl.ds(s * RPT, RPT)])
    wait_idx(0)
    fire_gather(0)
    wait_idx(1)
    fire_gather(1)
    plsc.subcore_barrier()

    def step(j, b):
        wait_idx((b + 2) % _NBUF)
        fire_gather((b + 2) % _NBUF)
        wait_gather(b)
        pltpu.sync_copy(rows[b], acc.at[dst_b[b]], add=True)
        fire_idx(j + _NBUF, b)

    def body(k, carry):
        j0 = k * _NBUF
        for b in range(_NBUF):
            step(j0 + b, b)
        return carry

    n_main = (STEPS - 4) // _NBUF
    lax.fori_loop(0, n_main, body, 0, unroll=False)
    for j in range(n_main * _NBUF, STEPS):
        b = j % _NBUF
        if j + 2 < STEPS:
            wait_idx((b + 2) % _NBUF)
            fire_gather((b + 2) % _NBUF)
        wait_gather(b)
        pltpu.sync_copy(rows[b], acc.at[dst_b[b]], add=True)
        if j + _NBUF < STEPS:
            fire_idx(j + _NBUF, b)
    plsc.subcore_barrier()
    pltpu.sync_copy(acc.at[pl.ds(s * RPT, RPT)],
                    out_hbm.at[c].at[pl.ds(s * RPT, RPT)])


_sc_scatter_rows = functools.partial(
    pl.kernel, out_type=_SCATTER_OUT, mesh=_MESH,
    scratch_types=_SCATTER_SCRATCH)(_sc_scatter_rows_body)


_DEG_OUT = jax.ShapeDtypeStruct((NC, ACC_ROWS, D), jnp.float32)
_DEG_SCRATCH = [
    [pltpu.VMEM((LANES,), jnp.int32) for _ in range(_NBUF)],
    pltpu.VMEM((LANES, D), jnp.float32),
    pltpu.MemorySpace.VMEM_SHARED((ACC_ROWS, D), jnp.float32),
    [pltpu.SemaphoreType.DMA for _ in range(_NBUF)],
]


def _sc_degree_body(dst_hbm, ones_hbm, z_hbm, out_hbm, dst_b, ones_v, acc,
                    isems):
    c = lax.axis_index("c")
    s = lax.axis_index("s")
    wid = s * NC + c
    base = wid * EPT

    def fire_idx(j, b):
        pltpu.async_copy(dst_hbm.at[pl.ds(base + j * LANES, LANES)],
                         dst_b[b], isems[b])

    def wait_idx(b):
        pltpu.make_async_copy(dst_hbm.at[pl.ds(base, LANES)],
                              dst_b[b], isems[b]).wait()

    for b in range(_NBUF):
        fire_idx(b, b)
    pltpu.sync_copy(z_hbm, acc.at[pl.ds(s * RPT, RPT)])
    pltpu.sync_copy(ones_hbm, ones_v)
    plsc.subcore_barrier()

    def body(k, carry):
        j0 = k * _NBUF
        for b in range(_NBUF):
            wait_idx(b)
            pltpu.sync_copy(ones_v, acc.at[dst_b[b]], add=True)
            fire_idx(j0 + b + _NBUF, b)
        return carry

    n_main = (STEPS - 3) // _NBUF
    lax.fori_loop(0, n_main, body, 0, unroll=False)
    for j in range(n_main * _NBUF, STEPS):
        b = j % _NBUF
        wait_idx(b)
        pltpu.sync_copy(ones_v, acc.at[dst_b[b]], add=True)
        if j + _NBUF < STEPS:
            fire_idx(j + _NBUF, b)
    plsc.subcore_barrier()
    pltpu.sync_copy(acc.at[pl.ds(s * RPT, RPT)],
                    out_hbm.at[c].at[pl.ds(s * RPT, RPT)])


_sc_degree = functools.partial(
    pl.kernel, out_type=_DEG_OUT, mesh=_MESH,
    scratch_types=_DEG_SCRATCH)(_sc_degree_body)


_BLK = 1000


def _deg_spec():
    return pl.BlockSpec((NC, _BLK, D), lambda i: (0, i, 0))


def _s_spec():
    return pl.BlockSpec((NC, _BLK, D), lambda i: (0, i, 0))


def _dinv_block(degp_ref):
    deg = degp_ref[0, :, 0:1] + degp_ref[1, :, 0:1] + 1.0
    return lax.rsqrt(deg)


def _tc_g1_body(degp_ref, x_ref, w_ref, g_ref):
    dinv = _dinv_block(degp_ref)
    g_ref[...] = dinv * jnp.dot(x_ref[...], w_ref[...],
                                preferred_element_type=jnp.float32)


def _tc_g1(degp, x, W1):
    return pl.pallas_call(
        _tc_g1_body,
        grid=(N // _BLK,),
        in_specs=[
            _deg_spec(),
            pl.BlockSpec((_BLK, D), lambda i: (i, 0)),
            pl.BlockSpec((D, D), lambda i: (0, 0)),
        ],
        out_specs=pl.BlockSpec((_BLK, D), lambda i: (i, 0)),
        out_shape=jax.ShapeDtypeStruct((N, D), jnp.float32),
    )(degp, x, W1)


def _tc_g2_body(degp_ref, s_ref, g1_ref, b1_ref, w2_ref, g2_ref):
    dinv = _dinv_block(degp_ref)
    h1 = dinv * (s_ref[0] + s_ref[1] + g1_ref[...]) + b1_ref[...]
    h1 = jnp.maximum(h1, 0.0)
    g2_ref[...] = dinv * jnp.dot(h1, w2_ref[...],
                                 preferred_element_type=jnp.float32)


def _tc_g2(degp, S1, g1, b1r, W2):
    return pl.pallas_call(
        _tc_g2_body,
        grid=(N // _BLK,),
        in_specs=[
            _deg_spec(),
            _s_spec(),
            pl.BlockSpec((_BLK, D), lambda i: (i, 0)),
            pl.BlockSpec((1, D), lambda i: (0, 0)),
            pl.BlockSpec((D, D), lambda i: (0, 0)),
        ],
        out_specs=pl.BlockSpec((_BLK, D), lambda i: (i, 0)),
        out_shape=jax.ShapeDtypeStruct((N, D), jnp.float32),
    )(degp, S1, g1, b1r, W2)


def _tc_head_body(degp_ref, s_ref, g2_ref, b2_ref,
                  batch_ref, wm1_ref, bm1_ref, wm2_ref, bm2_ref, out_ref):
    deg = degp_ref[0, :, 0:1] + degp_ref[1, :, 0:1] + 1.0
    dinv = lax.rsqrt(deg)
    h2 = dinv * (s_ref[0] + s_ref[1] + g2_ref[...]) + b2_ref[...]
    gid = lax.broadcasted_iota(jnp.int32, (NG, N), 0).astype(jnp.float32)
    onehot = (batch_ref[...] == gid).astype(jnp.float32)
    sums = jnp.dot(onehot, h2, preferred_element_type=jnp.float32)
    counts = jnp.sum(onehot, axis=1, keepdims=True)
    pooled = sums / jnp.maximum(counts, 1.0)
    z = jnp.dot(pooled, wm1_ref[...], preferred_element_type=jnp.float32)
    z = jnp.maximum(z + bm1_ref[...], 0.0)
    out_ref[...] = (jnp.sum(z * wm2_ref[...], axis=1, keepdims=True)
                    + bm2_ref[...])


def _tc_head(degp, S2, g2, b2r, batchf, Wm1, bm1r, wm2r, bm2r):
    return pl.pallas_call(
        _tc_head_body,
        grid=(1,),
        in_specs=[
            pl.BlockSpec((NC, N, D), lambda i: (0, 0, 0)),
            pl.BlockSpec((NC, N, D), lambda i: (0, 0, 0)),
            pl.BlockSpec((N, D), lambda i: (0, 0)),
            pl.BlockSpec((1, D), lambda i: (0, 0)),
            pl.BlockSpec((1, N), lambda i: (0, 0)),
            pl.BlockSpec((D, 16), lambda i: (0, 0)),
            pl.BlockSpec((1, 16), lambda i: (0, 0)),
            pl.BlockSpec((1, 16), lambda i: (0, 0)),
            pl.BlockSpec((1, 1), lambda i: (0, 0)),
        ],
        out_specs=pl.BlockSpec((NG, 1), lambda i: (0, 0)),
        out_shape=jax.ShapeDtypeStruct((NG, 1), jnp.float32),
    )(degp, S2, g2, b2r, batchf, Wm1, bm1r, wm2r, bm2r)


def kernel(x, edge_index, batch, W1, b1, W2, b2, Wm1, bm1, Wm2, bm2):
    src = edge_index[0]
    dst = edge_index[1]
    pad = E_PAD - E
    it = jnp.arange(pad, dtype=jnp.int32)
    src_p = jnp.concatenate([src, it % N])
    dst_p = jnp.concatenate([dst, TRASH + (it % N_TRASH)])
    zeros_d = jnp.zeros((RPT, D), jnp.float32)
    ones_d = jnp.ones((LANES, D), jnp.float32)

    degp = _sc_degree(dst_p, ones_d, zeros_d)

    g1 = _tc_g1(degp, x, W1)
    S1 = _sc_scatter_rows(g1, src_p, dst_p, zeros_d)
    g2 = _tc_g2(degp, S1, g1, b1.reshape(1, D), W2)
    S2 = _sc_scatter_rows(g2, src_p, dst_p, zeros_d)
    out = _tc_head(degp, S2, g2,
                   b2.reshape(1, D),
                   batch.astype(jnp.float32).reshape(1, N),
                   Wm1, bm1.reshape(1, 16),
                   Wm2.reshape(1, 16), bm2.reshape(1, 1))
    return out.reshape(-1)


# speedup vs baseline: 4.0775x; 1.1866x over previous
import functools

import jax
import jax.numpy as jnp
from jax import lax
from jax.experimental import pallas as pl
from jax.experimental.pallas import tpu as pltpu
from jax.experimental.pallas import tpu_sc as plsc

N = 10000
E = 320000
D = 128
NG = 64

NC = 2
NS = 16
NW = NC * NS

LANES = 128
STEPS = 80
EPT = STEPS * LANES
E_PAD = NW * EPT
ACC_ROWS = 10112
RPT = ACC_ROWS // NS
TRASH = N
N_TRASH = ACC_ROWS - N

_MESH = plsc.VectorSubcoreMesh(core_axis_name="c", subcore_axis_name="s")


_NBUF = 3
_SCATTER_OUT = jax.ShapeDtypeStruct((NC, ACC_ROWS, D), jnp.float32)
_SCATTER_SCRATCH = [
    [pltpu.VMEM((LANES,), jnp.int32) for _ in range(_NBUF)],
    [pltpu.VMEM((LANES,), jnp.int32) for _ in range(_NBUF)],
    [pltpu.VMEM((LANES, D), jnp.float32) for _ in range(_NBUF)],
    pltpu.MemorySpace.VMEM_SHARED((ACC_ROWS, D), jnp.float32),
    [pltpu.SemaphoreType.DMA for _ in range(_NBUF)],
    [pltpu.SemaphoreType.DMA for _ in range(_NBUF)],
]


def _sc_scatter_rows_body(g_hbm, src_hbm, dst_hbm, z_hbm, out_hbm,
                          src_b, dst_b, rows, acc, isems, gsems):
    c = lax.axis_index("c")
    s = lax.axis_index("s")
    wid = s * NC + c
    base = wid * EPT

    def fire_idx(j, b):
        pltpu.async_copy(src_hbm.at[pl.ds(base + j * LANES, LANES)],
                         src_b[b], isems[b])
        pltpu.async_copy(dst_hbm.at[pl.ds(base + j * LANES, LANES)],
                         dst_b[b], isems[b])

    def wait_idx(b):
        pltpu.make_async_copy(src_hbm.at[pl.ds(base, LANES)],
                              src_b[b], isems[b]).wait()
        pltpu.make_async_copy(dst_hbm.at[pl.ds(base, LANES)],
                              dst_b[b], isems[b]).wait()

    def fire_gather(b):
        pltpu.async_copy(g_hbm.at[src_b[b]], rows[b], gsems[b])

    def wait_gather(b):
        pltpu.make_async_copy(g_hbm.at[src_b[b]], rows[b], gsems[b]).wait()

    for b in range(_NBUF):
        fire_idx(b, b)
    pltpu.sync_copy(z_hbm, acc.at[pl.ds(s * RPT, RPT)])
    wait_idx(0)
    fire_gather(0)
    wait_idx(1)
    fire_gather(1)
    plsc.subcore_barrier()

    def step(j, b):
        wait_idx((b + 2) % _NBUF)
        fire_gather((b + 2) % _NBUF)
        wait_gather(b)
        pltpu.sync_copy(rows[b], acc.at[dst_b[b]], add=True)
        fire_idx(j + _NBUF, b)

    def body(k, carry):
        j0 = k * _NBUF
        for b in range(_NBUF):
            step(j0 + b, b)
        return carry

    n_main = (STEPS - 4) // _NBUF
    lax.fori_loop(0, n_main, body, 0, unroll=False)
    for j in range(n_main * _NBUF, STEPS):
        b = j % _NBUF
        if j + 2 < STEPS:
            wait_idx((b + 2) % _NBUF)
            fire_gather((b + 2) % _NBUF)
        wait_gather(b)
        pltpu.sync_copy(rows[b], acc.at[dst_b[b]], add=True)
        if j + _NBUF < STEPS:
            fire_idx(j + _NBUF, b)
    plsc.subcore_barrier()
    pltpu.sync_copy(acc.at[pl.ds(s * RPT, RPT)],
                    out_hbm.at[c].at[pl.ds(s * RPT, RPT)])


_sc_scatter_rows = functools.partial(
    pl.kernel, out_type=_SCATTER_OUT, mesh=_MESH,
    scratch_types=_SCATTER_SCRATCH)(_sc_scatter_rows_body)


HIST = NW * STEPS * 4
_DEG_OUT = jax.ShapeDtypeStruct((NC, NS, HIST), jnp.float32)
_DEG_SCRATCH = [
    pltpu.VMEM((EPT,), jnp.int32),
    pltpu.VMEM((HIST,), jnp.float32),
]


def _sc_degree_body(dst_hbm, z_hbm, out_hbm, dst_v, hist):
    c = lax.axis_index("c")
    s = lax.axis_index("s")
    wid = s * NC + c
    pltpu.sync_copy(dst_hbm.at[pl.ds(wid * EPT, EPT)], dst_v)
    pltpu.sync_copy(z_hbm, hist)
    ones16 = jnp.ones((16,), jnp.float32)

    def body(i, carry):
        idx = dst_v[pl.ds(i * 16, 16)]
        plsc.addupdate_scatter(hist, [idx], ones16)
        return carry

    lax.fori_loop(0, EPT // 16, body, 0, unroll=False)
    pltpu.sync_copy(hist, out_hbm.at[c].at[s])


_sc_degree = functools.partial(
    pl.kernel, out_type=_DEG_OUT, mesh=_MESH,
    compiler_params=pltpu.CompilerParams(needs_layout_passes=False),
    scratch_types=_DEG_SCRATCH)(_sc_degree_body)


def _tc_dinv_body(degp_ref, dinv_ref):
    deg = jnp.sum(degp_ref[...], axis=(0, 1)) + 1.0
    row = lax.rsqrt(deg).reshape(1, HIST)
    col16 = lax.dot_general(row, jnp.ones((1, 16), jnp.float32),
                            (((0,), (0,)), ((), ())),
                            preferred_element_type=jnp.float32)
    dinv_ref[...] = col16[:N]


def _tc_dinv(degp):
    return pl.pallas_call(
        _tc_dinv_body,
        grid=(1,),
        in_specs=[pl.BlockSpec((NC, NS, HIST), lambda i: (0, 0, 0))],
        out_specs=pl.BlockSpec((N, 16), lambda i: (0, 0)),
        out_shape=jax.ShapeDtypeStruct((N, 16), jnp.float32),
    )(degp)


_BLK = 1000


def _deg_spec():
    return pl.BlockSpec((_BLK, 16), lambda i: (i, 0))


def _s_spec():
    return pl.BlockSpec((NC, _BLK, D), lambda i: (0, i, 0))


def _dinv_block(dinv_ref):
    return dinv_ref[:, 0:1]


def _tc_g1_body(degp_ref, x_ref, w_ref, g_ref):
    dinv = _dinv_block(degp_ref)
    g_ref[...] = dinv * jnp.dot(x_ref[...], w_ref[...],
                                preferred_element_type=jnp.float32)


def _tc_g1(degp, x, W1):
    return pl.pallas_call(
        _tc_g1_body,
        grid=(N // _BLK,),
        in_specs=[
            _deg_spec(),
            pl.BlockSpec((_BLK, D), lambda i: (i, 0)),
            pl.BlockSpec((D, D), lambda i: (0, 0)),
        ],
        out_specs=pl.BlockSpec((_BLK, D), lambda i: (i, 0)),
        out_shape=jax.ShapeDtypeStruct((N, D), jnp.float32),
    )(degp, x, W1)


def _tc_g2_body(degp_ref, s_ref, g1_ref, b1_ref, w2_ref, g2_ref):
    dinv = _dinv_block(degp_ref)
    h1 = dinv * (s_ref[0] + s_ref[1] + g1_ref[...]) + b1_ref[...]
    h1 = jnp.maximum(h1, 0.0)
    g2_ref[...] = dinv * jnp.dot(h1, w2_ref[...],
                                 preferred_element_type=jnp.float32)


def _tc_g2(degp, S1, g1, b1r, W2):
    return pl.pallas_call(
        _tc_g2_body,
        grid=(N // _BLK,),
        in_specs=[
            _deg_spec(),
            _s_spec(),
            pl.BlockSpec((_BLK, D), lambda i: (i, 0)),
            pl.BlockSpec((1, D), lambda i: (0, 0)),
            pl.BlockSpec((D, D), lambda i: (0, 0)),
        ],
        out_specs=pl.BlockSpec((_BLK, D), lambda i: (i, 0)),
        out_shape=jax.ShapeDtypeStruct((N, D), jnp.float32),
    )(degp, S1, g1, b1r, W2)


def _tc_head_body(degp_ref, s_ref, g2_ref, b2_ref,
                  batch_ref, wm1_ref, bm1_ref, wm2_ref, bm2_ref, out_ref):
    dinv = degp_ref[:, 0:1]
    h2 = dinv * (s_ref[0] + s_ref[1] + g2_ref[...]) + b2_ref[...]
    gid = lax.broadcasted_iota(jnp.int32, (NG, N), 0).astype(jnp.float32)
    onehot = (batch_ref[...] == gid).astype(jnp.float32)
    sums = jnp.dot(onehot, h2, preferred_element_type=jnp.float32)
    counts = jnp.sum(onehot, axis=1, keepdims=True)
    pooled = sums / jnp.maximum(counts, 1.0)
    z = jnp.dot(pooled, wm1_ref[...], preferred_element_type=jnp.float32)
    z = jnp.maximum(z + bm1_ref[...], 0.0)
    out_ref[...] = (jnp.sum(z * wm2_ref[...], axis=1, keepdims=True)
                    + bm2_ref[...])


def _tc_head(degp, S2, g2, b2r, batchf, Wm1, bm1r, wm2r, bm2r):
    return pl.pallas_call(
        _tc_head_body,
        grid=(1,),
        in_specs=[
            pl.BlockSpec((N, 16), lambda i: (0, 0)),
            pl.BlockSpec((NC, N, D), lambda i: (0, 0, 0)),
            pl.BlockSpec((N, D), lambda i: (0, 0)),
            pl.BlockSpec((1, D), lambda i: (0, 0)),
            pl.BlockSpec((1, N), lambda i: (0, 0)),
            pl.BlockSpec((D, 16), lambda i: (0, 0)),
            pl.BlockSpec((1, 16), lambda i: (0, 0)),
            pl.BlockSpec((1, 16), lambda i: (0, 0)),
            pl.BlockSpec((1, 1), lambda i: (0, 0)),
        ],
        out_specs=pl.BlockSpec((NG, 1), lambda i: (0, 0)),
        out_shape=jax.ShapeDtypeStruct((NG, 1), jnp.float32),
    )(degp, S2, g2, b2r, batchf, Wm1, bm1r, wm2r, bm2r)


def kernel(x, edge_index, batch, W1, b1, W2, b2, Wm1, bm1, Wm2, bm2):
    src = edge_index[0]
    dst = edge_index[1]
    pad = E_PAD - E
    it = jnp.arange(pad, dtype=jnp.int32)
    src_p = jnp.concatenate([src, it % N])
    dst_p = jnp.concatenate([dst, TRASH + (it % N_TRASH)])
    zeros_d = jnp.zeros((RPT, D), jnp.float32)
    zeros_h = jnp.zeros((HIST,), jnp.float32)

    degp = _sc_degree(dst_p, zeros_h)
    dinv = _tc_dinv(degp)

    g1 = _tc_g1(dinv, x, W1)
    S1 = _sc_scatter_rows(g1, src_p, dst_p, zeros_d)
    g2 = _tc_g2(dinv, S1, g1, b1.reshape(1, D), W2)
    S2 = _sc_scatter_rows(g2, src_p, dst_p, zeros_d)
    out = _tc_head(dinv, S2, g2,
                   b2.reshape(1, D),
                   batch.astype(jnp.float32).reshape(1, N),
                   Wm1, bm1.reshape(1, 16),
                   Wm2.reshape(1, 16), bm2.reshape(1, 1))
    return out.reshape(-1)
